# baseline (speedup 1.0000x reference)
import functools

import jax
import jax.numpy as jnp
from jax import lax
from jax.experimental import pallas as pl
from jax.experimental.pallas import tpu as pltpu
from jax.experimental.pallas import tpu_sc as plsc

_SMALL = 1e-8
_BN_EPS = 1e-5
_D = 64
_NC = 2
_NS = 16
_NW = _NC * _NS
_L16 = 16


def _make_pool(B, Lp):
    nch = Lp // _L16
    spw = B // _NW
    chunks = []
    off = 0
    while off < Lp:
        cl = min(128, Lp - off)
        chunks.append((off, cl))
        off += cl

    mesh = plsc.VectorSubcoreMesh(core_axis_name="c", subcore_axis_name="s")

    @functools.partial(
        pl.kernel,
        out_type=jax.ShapeDtypeStruct((B, _D), jnp.float32),
        mesh=mesh,
        compiler_params=pltpu.CompilerParams(
            needs_layout_passes=False, use_tc_tiling_on_sc=False),
        scratch_types=[
            pltpu.VMEM((Lp,), jnp.int32),
            pltpu.VMEM((Lp, _D), jnp.float32),
            pltpu.VMEM((Lp,), jnp.float32),
            pltpu.VMEM((Lp,), jnp.float32),
            pltpu.VMEM((_D,), jnp.float32),
            pltpu.VMEM((spw, _D), jnp.float32),
            pltpu.SemaphoreType.DMA,
        ],
    )
    def pool(seqs, table, scale, bias, w, out,
             idx_v, rows_v, scale_v, bias_v, w_v, out_v, sem):
        wid = lax.axis_index("s") * _NC + lax.axis_index("c")
        base = wid * spw
        pltpu.sync_copy(scale, scale_v)
        pltpu.sync_copy(bias, bias_v)
        pltpu.sync_copy(w, w_v)

        def seq_body(s, _):
            wregs = [w_v[pl.ds(q * 16, 16)] for q in range(4)]
            pltpu.sync_copy(seqs.at[base + s], idx_v)
            cps = [
                pltpu.async_copy(table.at[idx_v.at[pl.ds(o, c)]],
                                 rows_v.at[pl.ds(o, c)], sem)
                for (o, c) in chunks
            ]
            for cp in cps:
                cp.wait()

            def chunk_body(c, carry):
                a0, a1, a2, a3, wsv = carry
                lanes = lax.iota(jnp.int32, _L16)
                dots = jnp.zeros((_L16,), jnp.float32)
                for k in range(_L16):
                    l = c * _L16 + k
                    m = rows_v[l, pl.ds(0, 16)] * wregs[0]
                    for q in range(1, 4):
                        m = m + rows_v[l, pl.ds(q * 16, 16)] * wregs[q]
                    dots = jnp.where(lanes == k, jnp.sum(m), dots)
                ids = idx_v[pl.ds(c * _L16, _L16)]
                x = dots * scale_v[pl.ds(c * _L16, _L16)] \
                    + bias_v[pl.ds(c * _L16, _L16)]
                sig = 1.0 / (1.0 + jnp.exp(-x))
                wv16 = jnp.where(ids != 0, sig, 0.0)
                accs = [a0, a1, a2, a3]
                for k in range(_L16):
                    l = c * _L16 + k
                    ws = wv16[k]
                    for q in range(4):
                        accs[q] = accs[q] + ws * rows_v[l, pl.ds(q * 16, 16)]
                return (accs[0], accs[1], accs[2], accs[3], wsv + wv16)

            z = jnp.zeros((_L16,), jnp.float32)
            a0, a1, a2, a3, wsv = lax.fori_loop(
                0, nch, chunk_body, (z, z, z, z, z))
            r = 1.0 / (jnp.zeros((_L16,), jnp.float32) + jnp.sum(wsv) + _SMALL)
            out_v[s, pl.ds(0, 16)] = a0 * r
            out_v[s, pl.ds(16, 16)] = a1 * r
            out_v[s, pl.ds(32, 16)] = a2 * r
            out_v[s, pl.ds(48, 16)] = a3 * r
            return 0

        lax.fori_loop(0, spw, seq_body, 0)
        pltpu.sync_copy(out_v, out.at[pl.ds(base, spw)])

    return pool


def _sim_body(q_ref, c_ref, o_ref):
    q = q_ref[...]
    c = c_ref[...]
    qn = q / (jnp.sqrt(jnp.sum(q * q, axis=1, keepdims=True)) + _SMALL)
    cn = c / (jnp.sqrt(jnp.sum(c * c, axis=1, keepdims=True)) + _SMALL)
    o_ref[...] = lax.dot_general(
        qn, cn, (((1,), (1,)), ((), ())),
        preferred_element_type=jnp.float32)


def _similarity(qm, cm):
    B = qm.shape[0]
    BQ, BC = 512, 1024
    return pl.pallas_call(
        _sim_body,
        grid=(B // BQ, B // BC),
        in_specs=[
            pl.BlockSpec((BQ, _D), lambda i, j: (i, 0)),
            pl.BlockSpec((BC, _D), lambda i, j: (j, 0)),
        ],
        out_specs=pl.BlockSpec((BQ, BC), lambda i, j: (i, j)),
        out_shape=jax.ShapeDtypeStruct((B, B), jnp.float32),
    )(qm, cm)


def _prep(seqs, gamma, beta, mean, var, W, L):
    Lp = ((L + _L16 - 1) // _L16) * _L16
    seqs_p = jnp.pad(seqs.astype(jnp.int32), ((0, 0), (0, Lp - L)))
    sc = gamma / jnp.sqrt(var + _BN_EPS)
    bs = beta - mean * sc
    sc_p = jnp.pad(sc, (0, Lp - L), constant_values=1.0)
    bs_p = jnp.pad(bs, (0, Lp - L))
    return seqs_p, sc_p, bs_p, W.reshape(-1), Lp


def kernel(code_seqs, query_seqs, code_table, query_table, Wc, Wq,
           gamma_c, beta_c, mean_c, var_c, gamma_q, beta_q, mean_q, var_q):
    B, LC = code_seqs.shape
    _, LQ = query_seqs.shape
    cs, csc, cbs, wc, LCp = _prep(code_seqs, gamma_c, beta_c, mean_c, var_c,
                                  Wc, LC)
    qs, qsc, qbs, wq, LQp = _prep(query_seqs, gamma_q, beta_q, mean_q, var_q,
                                  Wq, LQ)
    cmean = _make_pool(B, LCp)(cs, code_table, csc, cbs, wc)
    qmean = _make_pool(B, LQp)(qs, query_table, qsc, qbs, wq)
    return _similarity(qmean, cmean)
